# Initial kernel scaffold; baseline (speedup 1.0000x reference)
import jax
import jax.numpy as jnp
from jax.experimental import pallas as pl


def kernel(inputs, word_table, pos_table):
    raise NotImplementedError("write your pallas kernel here")


# speedup vs baseline: 1.4298x; 1.4298x over previous
import functools

import jax
import jax.numpy as jnp
from jax import lax
from jax.experimental import pallas as pl
from jax.experimental.pallas import tpu as pltpu
from jax.experimental.pallas import tpu_sc as plsc

_VOCAB = 1000000
_SEQ = 200
_DIM = 32
_BATCH = 4096
_N = _BATCH * _SEQ

_NC = 2
_NS = 16
_NW = _NC * _NS
_PER_W = _N // _NW
_G = 100
_CHUNK = 800
_GPC = _CHUNK // _G
_NCHUNK = _PER_W // _CHUNK
_REPS = _CHUNK // _SEQ
_IDX_ROWS = _PER_W // _G


def _body(idx_hbm, word_hbm, pos_hbm, out_hbm,
          idx_v, rows0, rows1, pos_v, gsem0, gsem1, osem0, osem1):
    wid = lax.axis_index("s") * _NC + lax.axis_index("c")
    idx_row0 = wid * _IDX_ROWS
    out_row0 = wid * _PER_W

    rows = (rows0, rows1)
    gsem = (gsem0, gsem1)
    osem = (osem0, osem1)

    pltpu.sync_copy(idx_hbm.at[pl.ds(idx_row0, _IDX_ROWS)], idx_v)
    pltpu.sync_copy(pos_hbm, pos_v)

    def start_gather(c, b):
        for g in range(_GPC):
            pltpu.make_async_copy(
                word_hbm.at[idx_v.at[c * _GPC + g]],
                rows[b].at[pl.ds(g * _G, _G)],
                gsem[b],
            ).start()

    def wait_gather(c, b):
        for g in range(_GPC):
            pltpu.make_async_copy(
                word_hbm.at[idx_v.at[c * _GPC + g]],
                rows[b].at[pl.ds(g * _G, _G)],
                gsem[b],
            ).wait()

    def out_copy(c, b):
        return pltpu.make_async_copy(
            rows[b],
            out_hbm.at[pl.ds(out_row0 + c * _CHUNK, _CHUNK)],
            osem[b],
        )

    def add_pos(b):
        rbuf = rows[b]

        def body(p, carry):
            for h in range(2):
                pv = pos_v[p, pl.ds(h * 16, 16)]
                for rep in range(_REPS):
                    r = rep * _SEQ + p
                    rbuf[r, pl.ds(h * 16, 16)] = rbuf[r, pl.ds(h * 16, 16)] + pv
            return carry

        lax.fori_loop(0, _SEQ, body, 0, unroll=2)

    start_gather(0, 0)

    def chunk_iter(t, carry):
        for b in range(2):
            c = t * 2 + b
            nb = 1 - b

            @pl.when(c + 1 < _NCHUNK)
            def _prefetch():
                @pl.when(c >= 1)
                def _drain_prev_out():
                    out_copy(c - 1, nb).wait()

                start_gather(c + 1, nb)

            wait_gather(c, b)
            add_pos(b)
            out_copy(c, b).start()
        return carry

    lax.fori_loop(0, _NCHUNK // 2, chunk_iter, 0)

    out_copy(_NCHUNK - 2, 0).wait()
    out_copy(_NCHUNK - 1, 1).wait()


@jax.jit
def _embed(idx2d, word_table, pos_table):
    mesh = plsc.VectorSubcoreMesh(core_axis_name="c", subcore_axis_name="s")
    return pl.kernel(
        _body,
        out_type=jax.ShapeDtypeStruct((_N, _DIM), jnp.float32),
        mesh=mesh,
        scratch_types=[
            pltpu.VMEM((_IDX_ROWS, _G), jnp.int32),
            pltpu.VMEM((_CHUNK, _DIM), jnp.float32),
            pltpu.VMEM((_CHUNK, _DIM), jnp.float32),
            pltpu.VMEM((_SEQ, _DIM), jnp.float32),
            pltpu.SemaphoreType.DMA,
            pltpu.SemaphoreType.DMA,
            pltpu.SemaphoreType.DMA,
            pltpu.SemaphoreType.DMA,
        ],
        compiler_params=pltpu.CompilerParams(use_tc_tiling_on_sc=False),
    )(idx2d, word_table, pos_table)


def kernel(inputs, word_table, pos_table):
    idx2d = inputs.reshape(_N // _G, _G)
    out = _embed(idx2d, word_table, pos_table)
    return out.reshape(_BATCH, _SEQ, _DIM)


# speedup vs baseline: 1.4340x; 1.0029x over previous
import jax
import jax.numpy as jnp
from jax import lax
from jax.experimental import pallas as pl
from jax.experimental.pallas import tpu as pltpu
from jax.experimental.pallas import tpu_sc as plsc

_VOCAB = 1000000
_SEQ = 200
_DIM = 32
_BATCH = 4096

_NC = 2
_NS = 16
_NW = _NC * _NS
_BPW = _BATCH // _NW
_CB = 4
_NCHUNK = _BPW // _CB


def _body(idx_hbm, word_hbm, pos_hbm, out_hbm,
          idx_v, rows0, rows1, pos_v, gsem0, gsem1, osem0, osem1):
    wid = lax.axis_index("s") * _NC + lax.axis_index("c")
    gb0 = wid * _BPW

    rows = (rows0, rows1)
    gsem = (gsem0, gsem1)
    osem = (osem0, osem1)

    pltpu.sync_copy(idx_hbm.at[pl.ds(gb0, _BPW)], idx_v)
    pltpu.sync_copy(pos_hbm, pos_v)

    def start_gather(c, b):
        for bi in range(_CB):
            pltpu.make_async_copy(
                word_hbm.at[idx_v.at[c * _CB + bi]],
                rows[b].at[bi],
                gsem[b],
            ).start()

    def wait_gather(c, b):
        for bi in range(_CB):
            pltpu.make_async_copy(
                word_hbm.at[idx_v.at[c * _CB + bi]],
                rows[b].at[bi],
                gsem[b],
            ).wait()

    def out_copy(c, b):
        return pltpu.make_async_copy(
            rows[b],
            out_hbm.at[pl.ds(gb0 + c * _CB, _CB)],
            osem[b],
        )

    def add_pos(b):
        rbuf = rows[b]

        def body(p, carry):
            for h in range(2):
                pv = pos_v[p, pl.ds(h * 16, 16)]
                for bi in range(_CB):
                    rbuf[bi, p, pl.ds(h * 16, 16)] = (
                        rbuf[bi, p, pl.ds(h * 16, 16)] + pv)
            return carry

        lax.fori_loop(0, _SEQ, body, 0, unroll=2)

    start_gather(0, 0)

    def chunk_iter(t, carry):
        for b in range(2):
            c = t * 2 + b
            nb = 1 - b

            @pl.when(c + 1 < _NCHUNK)
            def _prefetch():
                @pl.when(c >= 1)
                def _drain_prev_out():
                    out_copy(c - 1, nb).wait()

                start_gather(c + 1, nb)

            wait_gather(c, b)
            add_pos(b)
            out_copy(c, b).start()
        return carry

    lax.fori_loop(0, _NCHUNK // 2, chunk_iter, 0)

    out_copy(_NCHUNK - 2, 0).wait()
    out_copy(_NCHUNK - 1, 1).wait()


@jax.jit
def _embed(inputs, word_table, pos_table):
    mesh = plsc.VectorSubcoreMesh(core_axis_name="c", subcore_axis_name="s")
    return pl.kernel(
        _body,
        out_type=jax.ShapeDtypeStruct((_BATCH, _SEQ, _DIM), jnp.float32),
        mesh=mesh,
        scratch_types=[
            pltpu.VMEM((_BPW, _SEQ), jnp.int32),
            pltpu.VMEM((_CB, _SEQ, _DIM), jnp.float32),
            pltpu.VMEM((_CB, _SEQ, _DIM), jnp.float32),
            pltpu.VMEM((_SEQ, _DIM), jnp.float32),
            pltpu.SemaphoreType.DMA,
            pltpu.SemaphoreType.DMA,
            pltpu.SemaphoreType.DMA,
            pltpu.SemaphoreType.DMA,
        ],
        compiler_params=pltpu.CompilerParams(use_tc_tiling_on_sc=False),
    )(inputs, word_table, pos_table)


def kernel(inputs, word_table, pos_table):
    return _embed(inputs, word_table, pos_table)
